# baseline (speedup 1.0000x reference)
import functools

import jax
import jax.numpy as jnp
from jax import lax
from jax.experimental import pallas as pl
from jax.experimental.pallas import tpu as pltpu
from jax.experimental.pallas import tpu_sc as plsc

B = 4096
D = 768
E = 8
D_ADAPT = 256
D_PROJ = 1024

BLK = 1024


def _logits_body(x_ref, Wr_ref, br_ref, out_ref):
    lg = (jnp.dot(x_ref[...], Wr_ref[...],
                  preferred_element_type=jnp.float32) + br_ref[...])
    out_ref[...] = lg.T


def _router_logits(x, W_router, b_router):
    return pl.pallas_call(
        _logits_body,
        grid=(4,),
        in_specs=[
            pl.BlockSpec((B // 4, D), lambda i: (i, 0)),
            pl.BlockSpec((D, E), lambda i: (0, 0)),
            pl.BlockSpec((1, E), lambda i: (0, 0)),
        ],
        out_specs=pl.BlockSpec((E, B // 4), lambda i: (0, i)),
        out_shape=jax.ShapeDtypeStruct((E, B), jnp.float32),
    )(x, W_router, b_router.reshape(1, E))


_NW = 32
_TOK_W = B // _NW
_GRP = _TOK_W // 16


def _route_body(logits_hbm, w_hbm, l_v, w_v):
    wid = lax.axis_index("s") * 2 + lax.axis_index("c")
    base = wid * _TOK_W
    for e in range(E):
        pltpu.sync_copy(logits_hbm.at[e, pl.ds(base, _TOK_W)],
                        l_v.at[e, :])
    i16 = jnp.int32
    for g in range(_GRP):
        sl = pl.ds(g * 16, 16)
        p = [l_v[e, sl] for e in range(E)]
        m = p[0]
        for e in range(1, E):
            m = jnp.maximum(m, p[e])
        p = [jnp.exp(v - m) for v in p]
        s = p[0]
        for e in range(1, E):
            s = s + p[e]
        p = [v / s for v in p]
        t1 = p[0]
        for e in range(1, E):
            t1 = jnp.maximum(t1, p[e])
        i1 = jnp.full((16,), E, dtype=i16)
        for e in range(E - 1, -1, -1):
            i1 = jnp.where(p[e] == t1, jnp.full((16,), e, dtype=i16), i1)
        neg = jnp.full((16,), jnp.float32(-3.0e38))
        p2 = [jnp.where(i1 == jnp.full((16,), e, dtype=i16), neg, p[e])
              for e in range(E)]
        t2 = p2[0]
        for e in range(1, E):
            t2 = jnp.maximum(t2, p2[e])
        i2 = jnp.full((16,), E, dtype=i16)
        for e in range(E - 1, -1, -1):
            i2 = jnp.where(p2[e] == t2, jnp.full((16,), e, dtype=i16), i2)
        zero = jnp.zeros((16,), jnp.float32)
        ev = [jnp.full((16,), e, dtype=i16) for e in range(E)]
        wv = [jnp.where((i1 == ev[e]) | (i2 == ev[e]), p[e], zero)
              for e in range(E)]
        s2 = wv[0]
        for e in range(1, E):
            s2 = s2 + wv[e]
        s2 = s2 + jnp.full((16,), jnp.float32(1e-9))
        for e in range(E):
            w_v[e, sl] = wv[e] / s2
    for e in range(E):
        pltpu.sync_copy(w_v.at[e, :],
                        w_hbm.at[e, pl.ds(base, _TOK_W)])


def _route_weights(logitsT):
    mesh = plsc.VectorSubcoreMesh(core_axis_name="c", subcore_axis_name="s")
    k = functools.partial(
        pl.kernel,
        mesh=mesh,
        out_type=jax.ShapeDtypeStruct((E, B), jnp.float32),
        scratch_types=[
            pltpu.VMEM((E, _TOK_W), jnp.float32),
            pltpu.VMEM((E, _TOK_W), jnp.float32),
        ],
    )(_route_body)
    return k(logitsT)


def _main_body(x_ref, wT_ref, We1_ref, be1_ref, We2_ref, be2_ref,
               Wp1_ref, bp1_ref, lng_ref, lnb_ref, Wp2_ref, bp2_ref, out_ref):
    f32 = jnp.float32
    x = x_ref[...]
    w = wT_ref[...].T
    sw = jnp.sum(w, axis=-1, keepdims=True)

    acc = jnp.zeros((BLK, D), dtype=f32)
    for e in range(E):
        h = jnp.dot(x, We1_ref[e], preferred_element_type=f32) + be1_ref[e]
        h = jnp.maximum(h, 0.0)
        eo = jnp.dot(h, We2_ref[e], preferred_element_type=f32) + be2_ref[e]
        acc = acc + w[:, e:e + 1] * eo
    fused = acc + sw * x

    p = jnp.dot(fused, Wp1_ref[...], preferred_element_type=f32) + bp1_ref[...]
    mu = jnp.mean(p, axis=-1, keepdims=True)
    var = jnp.mean((p - mu) ** 2, axis=-1, keepdims=True)
    p = (p - mu) / jnp.sqrt(var + 1e-5) * lng_ref[...] + lnb_ref[...]
    p = jnp.maximum(p, 0.0)
    out_ref[...] = (jnp.dot(p, Wp2_ref[...], preferred_element_type=f32)
                    + bp2_ref[...])


@jax.jit
def kernel(x, W_router, b_router, W_e1, b_e1, W_e2, b_e2,
           W_p1, b_p1, ln_g, ln_b, W_p2, b_p2):
    logitsT = _router_logits(x, W_router, b_router)
    wT = _route_weights(logitsT)

    grid = (B // BLK,)
    fixed = lambda shape: pl.BlockSpec(shape, lambda i: (0,) * len(shape))
    return pl.pallas_call(
        _main_body,
        grid=grid,
        in_specs=[
            pl.BlockSpec((BLK, D), lambda i: (i, 0)),
            pl.BlockSpec((E, BLK), lambda i: (0, i)),
            fixed((E, D, D_ADAPT)),
            fixed((E, 1, D_ADAPT)),
            fixed((E, D_ADAPT, D)),
            fixed((E, 1, D)),
            fixed((D, D_PROJ)),
            fixed((1, D_PROJ)),
            fixed((1, D_PROJ)),
            fixed((1, D_PROJ)),
            fixed((D_PROJ, D)),
            fixed((1, D)),
        ],
        out_specs=pl.BlockSpec((BLK, D), lambda i: (i, 0)),
        out_shape=jax.ShapeDtypeStruct((B, D), jnp.float32),
        compiler_params=pltpu.CompilerParams(
            dimension_semantics=("parallel",),
        ),
    )(x, wT,
      W_e1, b_e1.reshape(E, 1, D_ADAPT), W_e2, b_e2.reshape(E, 1, D),
      W_p1, b_p1.reshape(1, D_PROJ), ln_g.reshape(1, D_PROJ),
      ln_b.reshape(1, D_PROJ), W_p2, b_p2.reshape(1, D))


# speedup vs baseline: 1.2918x; 1.2918x over previous
import jax
import jax.numpy as jnp
from jax.experimental import pallas as pl
from jax.experimental.pallas import tpu as pltpu

B = 4096
D = 768
E = 8
D_ADAPT = 256
D_PROJ = 1024

BLK = 1024


def _body(x_ref, Wr_ref, br_ref, We1_ref, be1_ref, We2_ref, be2_ref,
          Wp1_ref, bp1_ref, lng_ref, lnb_ref, Wp2_ref, bp2_ref, out_ref):
    f32 = jnp.float32
    x = x_ref[...]

    logits = jnp.dot(x, Wr_ref[...], preferred_element_type=f32) + br_ref[...]
    mx = jnp.max(logits, axis=-1, keepdims=True)
    exl = jnp.exp(logits - mx)
    probs = exl / jnp.sum(exl, axis=-1, keepdims=True)

    idx = jax.lax.broadcasted_iota(jnp.int32, (BLK, E), 1)
    top1 = jnp.max(probs, axis=-1, keepdims=True)
    i1 = jnp.min(jnp.where(probs == top1, idx, E), axis=-1, keepdims=True)
    probs2 = jnp.where(idx == i1, -jnp.inf, probs)
    top2 = jnp.max(probs2, axis=-1, keepdims=True)
    i2 = jnp.min(jnp.where(probs2 == top2, idx, E), axis=-1, keepdims=True)
    mask = (idx == i1) | (idx == i2)
    w = jnp.where(mask, probs, 0.0)
    w = w / (jnp.sum(w, axis=-1, keepdims=True) + 1e-9)
    sw = jnp.sum(w, axis=-1, keepdims=True)

    acc = jnp.zeros((BLK, D), dtype=f32)
    for e in range(E):
        h = jnp.dot(x, We1_ref[e], preferred_element_type=f32) + be1_ref[e]
        h = jnp.maximum(h, 0.0)
        eo = jnp.dot(h, We2_ref[e], preferred_element_type=f32) + be2_ref[e]
        acc = acc + w[:, e:e + 1] * eo
    fused = acc + sw * x

    p = jnp.dot(fused, Wp1_ref[...], preferred_element_type=f32) + bp1_ref[...]
    mu = jnp.mean(p, axis=-1, keepdims=True)
    var = jnp.mean((p - mu) ** 2, axis=-1, keepdims=True)
    p = (p - mu) / jnp.sqrt(var + 1e-5) * lng_ref[...] + lnb_ref[...]
    p = jnp.maximum(p, 0.0)
    out_ref[...] = (jnp.dot(p, Wp2_ref[...], preferred_element_type=f32)
                    + bp2_ref[...])


@jax.jit
def kernel(x, W_router, b_router, W_e1, b_e1, W_e2, b_e2,
           W_p1, b_p1, ln_g, ln_b, W_p2, b_p2):
    grid = (B // BLK,)
    fixed = lambda shape: pl.BlockSpec(shape, lambda i: (0,) * len(shape))
    return pl.pallas_call(
        _body,
        grid=grid,
        in_specs=[
            pl.BlockSpec((BLK, D), lambda i: (i, 0)),
            fixed((D, E)),
            fixed((1, E)),
            fixed((E, D, D_ADAPT)),
            fixed((E, 1, D_ADAPT)),
            fixed((E, D_ADAPT, D)),
            fixed((E, 1, D)),
            fixed((D, D_PROJ)),
            fixed((1, D_PROJ)),
            fixed((1, D_PROJ)),
            fixed((1, D_PROJ)),
            fixed((D_PROJ, D)),
            fixed((1, D)),
        ],
        out_specs=pl.BlockSpec((BLK, D), lambda i: (i, 0)),
        out_shape=jax.ShapeDtypeStruct((B, D), jnp.float32),
        compiler_params=pltpu.CompilerParams(
            dimension_semantics=("parallel",),
        ),
    )(x, W_router, b_router.reshape(1, E),
      W_e1, b_e1.reshape(E, 1, D_ADAPT), W_e2, b_e2.reshape(E, 1, D),
      W_p1, b_p1.reshape(1, D_PROJ), ln_g.reshape(1, D_PROJ),
      ln_b.reshape(1, D_PROJ), W_p2, b_p2.reshape(1, D))
